# baseline (speedup 1.0000x reference)
import functools

import jax
import jax.numpy as jnp
from jax import lax
from jax.experimental import pallas as pl
from jax.experimental.pallas import tpu as pltpu
from jax.experimental.pallas import tpu_sc as plsc

_B = 16384
_EMB = 16
_HID = 32
_NC = 2
_NS = 16
_NW = _NC * _NS
_BPW = _B // _NW


def _sc_gather(user_table, users, item_table, items):
    mesh = plsc.VectorSubcoreMesh(core_axis_name="c", subcore_axis_name="s")

    @functools.partial(
        pl.kernel,
        mesh=mesh,
        compiler_params=pltpu.CompilerParams(use_tc_tiling_on_sc=False),
        out_type=(
            jax.ShapeDtypeStruct((_B, _EMB), jnp.float32),
            jax.ShapeDtypeStruct((_B, _EMB), jnp.float32),
        ),
        scratch_types=[
            pltpu.VMEM((_BPW,), jnp.int32),
            pltpu.VMEM((_BPW, _EMB), jnp.float32),
            pltpu.VMEM((_BPW,), jnp.int32),
            pltpu.VMEM((_BPW, _EMB), jnp.float32),
            pltpu.SemaphoreType.DMA,
            pltpu.SemaphoreType.DMA,
        ],
    )
    def k(ut_hbm, u_hbm, it_hbm, i_hbm, uo_hbm, io_hbm,
          uidx, urows, iidx, irows, usem, isem):
        wid = lax.axis_index("s") * _NC + lax.axis_index("c")
        base = wid * _BPW
        pltpu.sync_copy(u_hbm.at[pl.ds(base, _BPW)], uidx)
        pltpu.sync_copy(i_hbm.at[pl.ds(base, _BPW)], iidx)
        cu = pltpu.async_copy(ut_hbm.at[uidx], urows, usem)
        ci = pltpu.async_copy(it_hbm.at[iidx], irows, isem)
        cu.wait()
        pltpu.sync_copy(urows, uo_hbm.at[pl.ds(base, _BPW)])
        ci.wait()
        pltpu.sync_copy(irows, io_hbm.at[pl.ds(base, _BPW)])

    return k(user_table, users, item_table, items)


def _tc_dense(ue, ie, ctx, ctx_W, ctx_b, W1, b1, W2, b2):
    def body(ue_ref, ie_ref, ctx_ref, cw_ref, cb_ref, w1_ref, b1_ref,
             w2_ref, b2_ref, out_ref):
        ctx_e = jnp.dot(ctx_ref[...], cw_ref[...],
                        preferred_element_type=jnp.float32) + cb_ref[...]
        w1 = w1_ref[...]
        h = (jnp.dot(ue_ref[...], w1[0:_EMB],
                     preferred_element_type=jnp.float32)
             + jnp.dot(ie_ref[...], w1[_EMB:2 * _EMB],
                       preferred_element_type=jnp.float32)
             + jnp.dot(ctx_e, w1[2 * _EMB:3 * _EMB],
                       preferred_element_type=jnp.float32)
             + b1_ref[...])
        h = jnp.maximum(h, 0.0)
        out = jnp.dot(h, w2_ref[...],
                      preferred_element_type=jnp.float32) + b2_ref[...]
        out_ref[...] = out

    return pl.pallas_call(
        body,
        out_shape=jax.ShapeDtypeStruct((_B, 1), jnp.float32),
    )(ue, ie, ctx, ctx_W, ctx_b, W1, b1, W2, b2)


def kernel(users, items, context_features, user_table, item_table,
           ctx_W, ctx_b, W1, b1, W2, b2):
    ue, ie = _sc_gather(user_table, users.astype(jnp.int32),
                        item_table, items.astype(jnp.int32))
    out = _tc_dense(ue, ie, context_features, ctx_W, ctx_b, W1, b1, W2, b2)
    return out[:, 0]
